# Initial kernel scaffold; baseline (speedup 1.0000x reference)
import jax
import jax.numpy as jnp
from jax.experimental import pallas as pl


def kernel(x, stem_w, stem_shift, conv2_b0_c1_w, conv2_b0_c1_shift, conv2_b0_c2_w, conv2_b0_c2_shift, conv2_b1_c1_w, conv2_b1_c1_shift, conv2_b1_c2_w, conv2_b1_c2_shift, conv2_b2_c1_w, conv2_b2_c1_shift, conv2_b2_c2_w, conv2_b2_c2_shift, conv3_b0_c1_w, conv3_b0_c1_shift, conv3_b0_c2_w, conv3_b0_c2_shift, conv3_b0_proj_w, conv3_b0_proj_shift, conv3_b1_c1_w, conv3_b1_c1_shift, conv3_b1_c2_w, conv3_b1_c2_shift, conv3_b2_c1_w, conv3_b2_c1_shift, conv3_b2_c2_w, conv3_b2_c2_shift, conv3_b3_c1_w, conv3_b3_c1_shift, conv3_b3_c2_w, conv3_b3_c2_shift, conv4_b0_c1_w, conv4_b0_c1_shift, conv4_b0_c2_w, conv4_b0_c2_shift, conv4_b0_proj_w, conv4_b0_proj_shift, conv4_b1_c1_w, conv4_b1_c1_shift, conv4_b1_c2_w, conv4_b1_c2_shift, conv4_b2_c1_w, conv4_b2_c1_shift, conv4_b2_c2_w, conv4_b2_c2_shift, conv4_b3_c1_w, conv4_b3_c1_shift, conv4_b3_c2_w, conv4_b3_c2_shift, conv4_b4_c1_w, conv4_b4_c1_shift, conv4_b4_c2_w, conv4_b4_c2_shift, conv4_b5_c1_w, conv4_b5_c1_shift, conv4_b5_c2_w, conv4_b5_c2_shift, conv5_b0_c1_w, conv5_b0_c1_shift, conv5_b0_c2_w, conv5_b0_c2_shift, conv5_b0_proj_w, conv5_b0_proj_shift, conv5_b1_c1_w, conv5_b1_c1_shift, conv5_b1_c2_w, conv5_b1_c2_shift, conv5_b2_c1_w, conv5_b2_c1_shift, conv5_b2_c2_w, conv5_b2_c2_shift, fc_w, fc_shift):
    raise NotImplementedError("write your pallas kernel here")


# speedup vs baseline: 1.0167x; 1.0167x over previous
import functools

import jax
import jax.numpy as jnp
from jax.experimental import pallas as pl
from jax.experimental.pallas import tpu as pltpu

_VMEM_BYTES = 48 * 1024 * 1024


def _rup(x, m):
    return ((x + m - 1) // m) * m


def _inv_spec(shape):
    ndim = len(shape)
    index_map = lambda b, _n=ndim: (0,) * _n
    if hasattr(pl, "Buffered"):
        return pl.BlockSpec(shape, index_map, pipeline_mode=pl.Buffered(1))
    return pl.BlockSpec(shape, index_map)


def _stem_kernel(g_ref, w_ref, s_ref, o_ref):
    acc = jnp.zeros((12544, 128), jnp.float32) + s_ref[...]
    for s in range(4):
        acc = acc + jnp.dot(g_ref[pl.ds(s * 112, 12544), :], w_ref[s],
                            preferred_element_type=jnp.float32)
    y3 = acc.astype(jnp.bfloat16).reshape(112, 112, 128)

    neg = jnp.full((1, 112, 128), -jnp.inf, jnp.bfloat16)
    y4 = y3.reshape(56, 2, 112, 128)
    ev, od = y4[:, 0], y4[:, 1]
    od_up = jnp.concatenate([neg, od[:-1]], axis=0)
    vi = jnp.maximum(jnp.maximum(ev, od), od_up)

    v4 = vi.reshape(56, 56, 2, 128)
    evj, odj = v4[:, :, 0], v4[:, :, 1]
    negc = jnp.full((56, 1, 128), -jnp.inf, jnp.bfloat16)
    odj_up = jnp.concatenate([negc, odj[:, :-1]], axis=1)
    p = jnp.maximum(jnp.maximum(evj, odj), odj_up)

    zc = jnp.zeros((56, 1, 128), jnp.bfloat16)
    o_ref[...] = jnp.concatenate([zc, p, zc], axis=1).reshape(3248, 128)


def _stem(x, stem_w, stem_shift):
    n = x.shape[0]
    xb = x.astype(jnp.bfloat16)
    xp = jnp.pad(xb, ((0, 0), (0, 0), (3, 3), (3, 3)))
    cols = [jnp.transpose(xp[:, :, :, b:b + 223:2], (0, 2, 3, 1))
            for b in range(7)]
    gfull = jnp.concatenate(cols, axis=-1)
    g = jnp.concatenate([gfull[:, 0::2], gfull[:, 1::2]], axis=-1)
    g = g.reshape(n, 115 * 112, 42)

    w4 = stem_w.reshape(7, 7, 3, 128)
    ws = jnp.zeros((4, 42, 128), jnp.bfloat16)
    for s in range(4):
        ws = ws.at[s, :21].set(w4[2 * s].reshape(21, 128))
    for s in range(3):
        ws = ws.at[s, 21:].set(w4[2 * s + 1].reshape(21, 128))

    return pl.pallas_call(
        _stem_kernel,
        out_shape=jax.ShapeDtypeStruct((n, 3248, 128), jnp.bfloat16),
        grid=(n,),
        in_specs=[
            pl.BlockSpec((None, 115 * 112, 42), lambda b: (b, 0, 0)),
            _inv_spec((4, 42, 128)),
            _inv_spec((1, 128)),
        ],
        out_specs=pl.BlockSpec((None, 3248, 128), lambda b: (b, 0, 0)),
        compiler_params=pltpu.CompilerParams(
            dimension_semantics=("parallel",),
            vmem_limit_bytes=_VMEM_BYTES),
    )(g, ws, stem_shift)


def _conv3x3(z_ref, src, w_ref, M, P, Wp, cin):
    z_ref[pl.ds(P, M), pl.ds(0, cin)] = src
    acc = None
    for di in range(3):
        for dj in range(3):
            off = P + (di - 1) * Wp + (dj - 1)
            win = z_ref[pl.ds(off, M), pl.ds(0, cin)]
            d = jnp.dot(win, w_ref[di * 3 + dj],
                        preferred_element_type=jnp.float32)
            acc = d if acc is None else acc + d
    return acc


def _pool2x2_block(y, H, W, C):
    H2, W2 = H // 2, W // 2
    y3 = y.reshape(H, W + 2, C)[:, 1:W + 1, :]
    y4 = y3.reshape(H2, 2, W, C)
    t = jnp.maximum(y4[:, 0], y4[:, 1])
    t2 = t.reshape(H2, W2, 2, C)
    p = jnp.maximum(t2[:, :, 0], t2[:, :, 1])
    zc = jnp.zeros((H2, 1, C), p.dtype)
    return jnp.concatenate([zc, p, zc], axis=1).reshape(H2 * (W2 + 2), C)


def _stage_kernel(*refs, H, W, plan, mode):
    Wp = W + 2
    M = H * Wp
    P = Wp + 1

    it = iter(refs)
    x_ref = next(it)
    blk_refs = []
    for has_proj, cin, cout in plan:
        w1, s1, w2, s2 = next(it), next(it), next(it), next(it)
        pr = (next(it), next(it)) if has_proj else None
        blk_refs.append((w1, s1, w2, s2, pr))
    o_ref, z1_ref, z2_ref = next(it), next(it), next(it)

    col = jax.lax.broadcasted_iota(jnp.int32, (M, 1), 0) % Wp
    interior = jnp.logical_and(col >= 1, col <= W)

    z1_ref[...] = jnp.zeros_like(z1_ref)
    z2_ref[...] = jnp.zeros_like(z2_ref)

    x = x_ref[...]
    for (has_proj, cin, cout), (w1, s1, w2, s2, pr) in zip(plan, blk_refs):
        acc = _conv3x3(z1_ref, x, w1, M, P, Wp, cin) + s1[...]
        y1 = jnp.where(interior, jnp.maximum(acc, 0.0), 0.0)
        y1 = y1.astype(jnp.bfloat16)
        if pr is not None:
            idn = jnp.dot(x, pr[0][...],
                          preferred_element_type=jnp.float32) + pr[1][...]
        else:
            idn = x.astype(jnp.float32)
        acc2 = _conv3x3(z2_ref, y1, w2, M, P, Wp, cout) + s2[...] + idn
        x = jnp.where(interior, jnp.maximum(acc2, 0.0), 0.0)
        x = x.astype(jnp.bfloat16)

    if mode == "pool":
        o_ref[...] = _pool2x2_block(x, H, W, x.shape[-1])
    else:
        o_ref[...] = jnp.sum(x.astype(jnp.float32), axis=0,
                             keepdims=True) * (1.0 / 49.0)


def _stage(xb, blocks, H, W, mode):
    n = xb.shape[0]
    Wp = W + 2
    M = H * Wp
    P = Wp + 1
    plan = tuple((blk["proj"] is not None,
                  blk["w1"].shape[1], blk["w1"].shape[2]) for blk in blocks)
    cout = plan[-1][2]
    mz = _rup(M + 2 * P, 8)

    args = [xb]
    in_specs = [pl.BlockSpec((None, M, plan[0][1]), lambda b: (b, 0, 0))]
    for blk in blocks:
        for nm in ("w1", "s1", "w2", "s2"):
            args.append(blk[nm])
            in_specs.append(_inv_spec(blk[nm].shape))
        if blk["proj"] is not None:
            for a in blk["proj"]:
                args.append(a)
                in_specs.append(_inv_spec(a.shape))

    if mode == "pool":
        m2 = (H // 2) * (W // 2 + 2)
        out_shape = jax.ShapeDtypeStruct((n, m2, cout), jnp.bfloat16)
        out_spec = pl.BlockSpec((None, m2, cout), lambda b: (b, 0, 0))
    else:
        out_shape = jax.ShapeDtypeStruct((n, 1, cout), jnp.float32)
        out_spec = pl.BlockSpec((None, 1, cout), lambda b: (b, 0, 0))

    return pl.pallas_call(
        functools.partial(_stage_kernel, H=H, W=W, plan=plan, mode=mode),
        out_shape=out_shape,
        grid=(n,),
        in_specs=in_specs,
        out_specs=out_spec,
        scratch_shapes=[pltpu.VMEM((mz, cout), jnp.bfloat16),
                        pltpu.VMEM((mz, cout), jnp.bfloat16)],
        compiler_params=pltpu.CompilerParams(
            dimension_semantics=("parallel",),
            vmem_limit_bytes=_VMEM_BYTES),
    )(*args)


def _fc_kernel(x_ref, w_ref, s_ref, o_ref):
    o_ref[...] = jnp.dot(x_ref[...], w_ref[...],
                         preferred_element_type=jnp.float32) + s_ref[...]


def _fc(feat, fc_w, fc_shift):
    n = feat.shape[0]
    return pl.pallas_call(
        _fc_kernel,
        out_shape=jax.ShapeDtypeStruct((n, fc_w.shape[1]), jnp.float32),
    )(feat.astype(jnp.bfloat16), fc_w, fc_shift)


def kernel(x, stem_w, stem_shift, conv2_b0_c1_w, conv2_b0_c1_shift, conv2_b0_c2_w, conv2_b0_c2_shift, conv2_b1_c1_w, conv2_b1_c1_shift, conv2_b1_c2_w, conv2_b1_c2_shift, conv2_b2_c1_w, conv2_b2_c1_shift, conv2_b2_c2_w, conv2_b2_c2_shift, conv3_b0_c1_w, conv3_b0_c1_shift, conv3_b0_c2_w, conv3_b0_c2_shift, conv3_b0_proj_w, conv3_b0_proj_shift, conv3_b1_c1_w, conv3_b1_c1_shift, conv3_b1_c2_w, conv3_b1_c2_shift, conv3_b2_c1_w, conv3_b2_c1_shift, conv3_b2_c2_w, conv3_b2_c2_shift, conv3_b3_c1_w, conv3_b3_c1_shift, conv3_b3_c2_w, conv3_b3_c2_shift, conv4_b0_c1_w, conv4_b0_c1_shift, conv4_b0_c2_w, conv4_b0_c2_shift, conv4_b0_proj_w, conv4_b0_proj_shift, conv4_b1_c1_w, conv4_b1_c1_shift, conv4_b1_c2_w, conv4_b1_c2_shift, conv4_b2_c1_w, conv4_b2_c1_shift, conv4_b2_c2_w, conv4_b2_c2_shift, conv4_b3_c1_w, conv4_b3_c1_shift, conv4_b3_c2_w, conv4_b3_c2_shift, conv4_b4_c1_w, conv4_b4_c1_shift, conv4_b4_c2_w, conv4_b4_c2_shift, conv4_b5_c1_w, conv4_b5_c1_shift, conv4_b5_c2_w, conv4_b5_c2_shift, conv5_b0_c1_w, conv5_b0_c1_shift, conv5_b0_c2_w, conv5_b0_c2_shift, conv5_b0_proj_w, conv5_b0_proj_shift, conv5_b1_c1_w, conv5_b1_c1_shift, conv5_b1_c2_w, conv5_b1_c2_shift, conv5_b2_c1_w, conv5_b2_c1_shift, conv5_b2_c2_w, conv5_b2_c2_shift, fc_w, fc_shift):
    def blk(w1, s1, w2, s2, proj=None):
        return {"w1": w1, "s1": s1, "w2": w2, "s2": s2, "proj": proj}

    xb = _stem(x, stem_w, stem_shift)

    xb = _stage(xb, [
        blk(conv2_b0_c1_w, conv2_b0_c1_shift, conv2_b0_c2_w, conv2_b0_c2_shift),
        blk(conv2_b1_c1_w, conv2_b1_c1_shift, conv2_b1_c2_w, conv2_b1_c2_shift),
        blk(conv2_b2_c1_w, conv2_b2_c1_shift, conv2_b2_c2_w, conv2_b2_c2_shift),
    ], 56, 56, "pool")

    xb = _stage(xb, [
        blk(conv3_b0_c1_w, conv3_b0_c1_shift, conv3_b0_c2_w, conv3_b0_c2_shift,
            (conv3_b0_proj_w, conv3_b0_proj_shift)),
        blk(conv3_b1_c1_w, conv3_b1_c1_shift, conv3_b1_c2_w, conv3_b1_c2_shift),
        blk(conv3_b2_c1_w, conv3_b2_c1_shift, conv3_b2_c2_w, conv3_b2_c2_shift),
        blk(conv3_b3_c1_w, conv3_b3_c1_shift, conv3_b3_c2_w, conv3_b3_c2_shift),
    ], 28, 28, "pool")

    xb = _stage(xb, [
        blk(conv4_b0_c1_w, conv4_b0_c1_shift, conv4_b0_c2_w, conv4_b0_c2_shift,
            (conv4_b0_proj_w, conv4_b0_proj_shift)),
        blk(conv4_b1_c1_w, conv4_b1_c1_shift, conv4_b1_c2_w, conv4_b1_c2_shift),
        blk(conv4_b2_c1_w, conv4_b2_c1_shift, conv4_b2_c2_w, conv4_b2_c2_shift),
        blk(conv4_b3_c1_w, conv4_b3_c1_shift, conv4_b3_c2_w, conv4_b3_c2_shift),
        blk(conv4_b4_c1_w, conv4_b4_c1_shift, conv4_b4_c2_w, conv4_b4_c2_shift),
        blk(conv4_b5_c1_w, conv4_b5_c1_shift, conv4_b5_c2_w, conv4_b5_c2_shift),
    ], 14, 14, "pool")

    feat = _stage(xb, [
        blk(conv5_b0_c1_w, conv5_b0_c1_shift, conv5_b0_c2_w, conv5_b0_c2_shift,
            (conv5_b0_proj_w, conv5_b0_proj_shift)),
        blk(conv5_b1_c1_w, conv5_b1_c1_shift, conv5_b1_c2_w, conv5_b1_c2_shift),
        blk(conv5_b2_c1_w, conv5_b2_c1_shift, conv5_b2_c2_w, conv5_b2_c2_shift),
    ], 7, 7, "feat")

    logits = _fc(feat.reshape(feat.shape[0], 512), fc_w, fc_shift)
    return logits[:, :10]
